# baseline (speedup 1.0000x reference)
import functools

import jax
import jax.numpy as jnp
from jax import lax
from jax.experimental import pallas as pl
from jax.experimental.pallas import tpu as pltpu
from jax.experimental.pallas import tpu_sc as plsc

_NW = 32
_CH = 128
_TE = 2048
_NB = 64


def _node_init_body(x_ref, w_ref, b_ref, o_ref):
    o_ref[...] = jnp.maximum(
        jnp.dot(x_ref[...], w_ref[...], preferred_element_type=jnp.float32)
        + b_ref[...], 0.0)


def _slab_body(ea_ref, w1t_ref, b1_ref, w2t_ref, o_ref):
    h1 = jnp.maximum(
        jnp.dot(ea_ref[...], w1t_ref[...], preferred_element_type=jnp.float32)
        + b1_ref[...], 0.0).astype(jnp.bfloat16)
    o_ref[...] = jnp.dot(h1, w2t_ref[...],
                         preferred_element_type=jnp.float32).astype(jnp.bfloat16)


def _msg_body(xs_ref, w_ref, b2r_ref, o_ref):
    te, d = xs_ref.shape
    w = w_ref[...]
    xs = xs_ref[...]
    acc = jnp.dot(xs, b2r_ref[...], preferred_element_type=jnp.float32)
    dpj = 128 // d
    nj = (d * d) // 128
    acc128 = None
    for j in range(nj):
        wj = w[:, 128 * j:128 * (j + 1)]
        xrep = jnp.concatenate(
            [jnp.broadcast_to(xs[:, dpj * j + a:dpj * j + a + 1], (te, d))
             for a in range(dpj)], axis=1)
        term = xrep * wj
        acc128 = term if acc128 is None else acc128 + term
    for a in range(dpj):
        acc = acc + acc128[:, d * a:d * (a + 1)]
    o_ref[...] = acc


def _node_update_body(accm_ref, acco_ref, out_ref, root_ref, cb_ref,
                      wih_ref, whh_ref, bih_ref, bhh_ref, hnew_ref, *, n):
    aggr = accm_ref[0, :n, :] + accm_ref[1, :n, :]
    deg = jnp.maximum(acco_ref[0, :n, 0:1] + acco_ref[1, :n, 0:1], 1.0)
    aggr = aggr / deg
    h = out_ref[...]
    d = h.shape[1]
    m = jnp.maximum(
        jnp.dot(h, root_ref[...], preferred_element_type=jnp.float32)
        + aggr + cb_ref[...], 0.0)
    gi = jnp.dot(m, wih_ref[...], preferred_element_type=jnp.float32) + bih_ref[...]
    gh = jnp.dot(h, whh_ref[...], preferred_element_type=jnp.float32) + bhh_ref[...]
    r = jax.nn.sigmoid(gi[:, :d] + gh[:, :d])
    z = jax.nn.sigmoid(gi[:, d:2 * d] + gh[:, d:2 * d])
    nn_ = jnp.tanh(gi[:, 2 * d:] + r * gh[:, 2 * d:])
    hnew_ref[...] = (1.0 - z) * nn_ + z * h


def _set2set_body(out_ref, brow_ref, wih_ref, whh_ref, bias_ref,
                  l1w_ref, l1b_ref, l2w_ref, l2b_ref, y_ref, *, nb, steps):
    xo = out_ref[...]
    n, d = xo.shape
    brow = brow_ref[...]
    gid = lax.broadcasted_iota(jnp.int32, (nb, n), 0)
    msk = gid == brow
    mf = msk.astype(jnp.float32)
    q_star = jnp.zeros((nb, 2 * d), jnp.float32)
    hh = jnp.zeros((nb, d), jnp.float32)
    cc = jnp.zeros((nb, d), jnp.float32)
    for _ in range(steps):
        gates = (jnp.dot(q_star, wih_ref[...], preferred_element_type=jnp.float32)
                 + jnp.dot(hh, whh_ref[...], preferred_element_type=jnp.float32)
                 + bias_ref[...])
        ii = jax.nn.sigmoid(gates[:, :d])
        ff = jax.nn.sigmoid(gates[:, d:2 * d])
        gg = jnp.tanh(gates[:, 2 * d:3 * d])
        oo = jax.nn.sigmoid(gates[:, 3 * d:])
        cc = ff * cc + ii * gg
        hh = oo * jnp.tanh(cc)
        s = lax.dot_general(hh, xo, (((1,), (1,)), ((), ())),
                            preferred_element_type=jnp.float32)
        e = jnp.sum(mf * s, axis=0, keepdims=True)
        emax = jnp.max(jnp.where(msk, e, -3e38), axis=1, keepdims=True)
        emax = jnp.where(emax > -1e38, emax, 0.0)
        eg = jnp.sum(mf * emax, axis=0, keepdims=True)
        ex = jnp.exp(e - eg)
        den = jnp.sum(mf * ex, axis=1, keepdims=True)
        dg = jnp.sum(mf * den, axis=0, keepdims=True)
        a = ex / jnp.maximum(dg, 1e-16)
        r_read = jnp.dot(mf * a, xo, preferred_element_type=jnp.float32)
        q_star = jnp.concatenate([hh, r_read], axis=1)
    y = jnp.maximum(
        jnp.dot(q_star, l1w_ref[...], preferred_element_type=jnp.float32)
        + l1b_ref[...], 0.0)
    y_ref[...] = (jnp.dot(y, l2w_ref[...], preferred_element_type=jnp.float32)
                  + l2b_ref[...])


def _tc_node_init(x, wt, b):
    n = x.shape[0]
    dout = wt.shape[1]
    return pl.pallas_call(
        _node_init_body,
        out_shape=jax.ShapeDtypeStruct((n, dout), jnp.float32),
    )(x, wt, b)


def _tc_slab(ea, w1t, b1, w2t, te):
    e_pad, fa = ea.shape
    c_h = w1t.shape[1]
    dd = w2t.shape[1]
    return pl.pallas_call(
        _slab_body,
        grid=(e_pad // te,),
        in_specs=[pl.BlockSpec((te, fa), lambda i: (i, 0)),
                  pl.BlockSpec((fa, c_h), lambda i: (0, 0)),
                  pl.BlockSpec((1, c_h), lambda i: (0, 0)),
                  pl.BlockSpec((c_h, dd), lambda i: (0, 0))],
        out_specs=pl.BlockSpec((te, dd), lambda i: (i, 0)),
        out_shape=jax.ShapeDtypeStruct((e_pad, dd), jnp.bfloat16),
    )(ea, w1t, b1, w2t)


def _tc_msg(xs, slab, b2r, te):
    e_pad, d = xs.shape
    dd = slab.shape[1]
    return pl.pallas_call(
        _msg_body,
        grid=(e_pad // te,),
        in_specs=[pl.BlockSpec((te, d), lambda i: (i, 0)),
                  pl.BlockSpec((te, dd), lambda i: (i, 0)),
                  pl.BlockSpec((d, d), lambda i: (0, 0))],
        out_specs=pl.BlockSpec((te, d), lambda i: (i, 0)),
        out_shape=jax.ShapeDtypeStruct((e_pad, d), jnp.float32),
    )(xs, slab, b2r)


def _tc_node_update(accs, acc_ones, out, root, cb, wiht, whht, bih, bhh, n):
    d = out.shape[1]
    return pl.pallas_call(
        functools.partial(_node_update_body, n=n),
        out_shape=jax.ShapeDtypeStruct((n, d), jnp.float32),
    )(accs, acc_ones, out, root, cb, wiht, whht, bih, bhh)


def _tc_set2set(out, brow, wiht, whht, bias, l1wt, l1b, l2wt, l2b, nb, steps):
    return pl.pallas_call(
        functools.partial(_set2set_body, nb=nb, steps=steps),
        out_shape=jax.ShapeDtypeStruct((nb, 1), jnp.float32),
    )(out, brow, wiht, whht, bias, l1wt, l1b, l2wt, l2b)


def _sc_gather(table, idx3, e_pad, nch):
    d = table.shape[1]
    mesh = plsc.VectorSubcoreMesh(core_axis_name="c", subcore_axis_name="s")

    def body(tab_ref, idx_ref, out_ref, idx_v, rows_v, sem):
        c = lax.axis_index("c")
        s = lax.axis_index("s")
        wid = s * 2 + c
        pltpu.sync_copy(idx_ref.at[wid], idx_v)
        base = wid * nch * _CH

        def step(j, carry):
            pltpu.async_copy(tab_ref.at[idx_v.at[j]], rows_v, sem).wait()
            pltpu.sync_copy(rows_v, out_ref.at[pl.ds(base + j * _CH, _CH)])
            return carry

        lax.fori_loop(0, nch, step, 0)

    f = pl.kernel(body,
                  out_type=jax.ShapeDtypeStruct((e_pad, d), jnp.float32),
                  mesh=mesh,
                  compiler_params=pltpu.CompilerParams(use_tc_tiling_on_sc=False),
                  scratch_types=[pltpu.VMEM((nch, _CH), jnp.int32),
                                 pltpu.VMEM((_CH, d), jnp.float32),
                                 pltpu.SemaphoreType.DMA])
    return f(table, idx3)


def _sc_scatter(msg, idx3, zeros_acc, n_acc, nch):
    d = msg.shape[1]
    rp = n_acc // 16
    mesh = plsc.VectorSubcoreMesh(core_axis_name="c", subcore_axis_name="s")

    def body(msg_ref, idx_ref, z_ref, out_ref, idx_v, rows_v, acc_sh, sem):
        c = lax.axis_index("c")
        s = lax.axis_index("s")
        wid = s * 2 + c
        pltpu.sync_copy(z_ref.at[pl.ds(s * rp, rp)], acc_sh.at[pl.ds(s * rp, rp)])
        pltpu.sync_copy(idx_ref.at[wid], idx_v)
        plsc.subcore_barrier()
        base = wid * nch * _CH

        def step(j, carry):
            pltpu.sync_copy(msg_ref.at[pl.ds(base + j * _CH, _CH)], rows_v)
            pltpu.sync_copy(rows_v, acc_sh.at[idx_v.at[j]], add=True)
            return carry

        lax.fori_loop(0, nch, step, 0)
        plsc.subcore_barrier()
        pltpu.sync_copy(acc_sh.at[pl.ds(s * rp, rp)],
                        out_ref.at[c, pl.ds(s * rp, rp)])

    f = pl.kernel(body,
                  out_type=jax.ShapeDtypeStruct((2, n_acc, d), jnp.float32),
                  mesh=mesh,
                  compiler_params=pltpu.CompilerParams(use_tc_tiling_on_sc=False),
                  scratch_types=[pltpu.VMEM((nch, _CH), jnp.int32),
                                 pltpu.VMEM((_CH, d), jnp.float32),
                                 pltpu.VMEM_SHARED((n_acc, d), jnp.float32),
                                 pltpu.SemaphoreType.DMA])
    return f(msg, idx3, zeros_acc)


def kernel(x, edge_index, edge_attr, batch, lin0_w, lin0_b, mlp_w1, mlp_b1,
           mlp_w2, mlp_b2, conv_root, conv_bias, gru_w_ih, gru_w_hh, gru_b_ih,
           gru_b_hh, lstm_w_ih, lstm_w_hh, lstm_b_ih, lstm_b_hh, lin1_w,
           lin1_b, lin2_w, lin2_b):
    n, _ = x.shape
    e = edge_index.shape[1]
    d = lin0_w.shape[0]
    c_h = mlp_w1.shape[0]
    nb = _NB

    nch = -(-e // (_NW * _CH))
    e_pad = _NW * nch * _CH
    n_acc = ((n + 1) + 15) // 16 * 16

    src = edge_index[0]
    dst = edge_index[1]
    pad_e = e_pad - e
    src3 = jnp.concatenate([src, jnp.zeros((pad_e,), jnp.int32)]).reshape(
        _NW, nch, _CH)
    dst3 = jnp.concatenate([dst, jnp.full((pad_e,), n, jnp.int32)]).reshape(
        _NW, nch, _CH)
    ea_p = jnp.concatenate(
        [edge_attr, jnp.zeros((pad_e, edge_attr.shape[1]), jnp.float32)])
    zeros_acc = jnp.zeros((n_acc, d), jnp.float32)
    ones_msg = jnp.ones((e_pad, d), jnp.float32)
    brow = batch.reshape(1, n)

    out = _tc_node_init(x, lin0_w.T, lin0_b.reshape(1, d))
    slab = _tc_slab(ea_p, mlp_w1.T, mlp_b1.reshape(1, c_h),
                    mlp_w2.T.astype(jnp.bfloat16), _TE)
    acc_ones = _sc_scatter(ones_msg, dst3, zeros_acc, n_acc, nch)

    b2r = mlp_b2.reshape(d, d)
    wiht = gru_w_ih.T
    whht = gru_w_hh.T
    bih = gru_b_ih.reshape(1, 3 * d)
    bhh = gru_b_hh.reshape(1, 3 * d)
    for _ in range(3):
        xsrc = _sc_gather(out, src3, e_pad, nch)
        msg = _tc_msg(xsrc, slab, b2r, _TE)
        accs = _sc_scatter(msg, dst3, zeros_acc, n_acc, nch)
        out = _tc_node_update(accs, acc_ones, out, conv_root,
                              conv_bias.reshape(1, d), wiht, whht, bih, bhh, n)

    y = _tc_set2set(out, brow, lstm_w_ih.T, lstm_w_hh.T,
                    (lstm_b_ih + lstm_b_hh).reshape(1, 4 * d),
                    lin1_w.T, lin1_b.reshape(1, d), lin2_w.T,
                    lin2_b.reshape(1, 1), nb, 3)
    return y.reshape(-1)


# speedup vs baseline: 1.1463x; 1.1463x over previous
import functools

import jax
import jax.numpy as jnp
from jax import lax
from jax.experimental import pallas as pl
from jax.experimental.pallas import tpu as pltpu
from jax.experimental.pallas import tpu_sc as plsc

_NW = 32
_CH = 128
_TE = 2048
_NB = 64


def _node_init_body(x_ref, w_ref, b_ref, o_ref):
    o_ref[...] = jnp.maximum(
        jnp.dot(x_ref[...], w_ref[...], preferred_element_type=jnp.float32)
        + b_ref[...], 0.0)


def _h1_body(ea_ref, w_ref, b_ref, o_ref):
    o_ref[...] = jnp.maximum(
        jnp.dot(ea_ref[...], w_ref[...], preferred_element_type=jnp.float32)
        + b_ref[...], 0.0).astype(jnp.bfloat16)


def _msg_body(xs_ref, h1_ref, w2t_ref, b2r_ref, o_ref):
    te, d = xs_ref.shape
    w = jnp.dot(h1_ref[...], w2t_ref[...],
                preferred_element_type=jnp.float32).astype(jnp.bfloat16)
    xs = xs_ref[...]
    acc = jnp.dot(xs, b2r_ref[...], preferred_element_type=jnp.float32)
    dpj = 128 // d
    nj = (d * d) // 128
    acc128 = None
    for j in range(nj):
        wj = w[:, 128 * j:128 * (j + 1)]
        xrep = jnp.concatenate(
            [jnp.broadcast_to(xs[:, dpj * j + a:dpj * j + a + 1], (te, d))
             for a in range(dpj)], axis=1)
        term = xrep * wj
        acc128 = term if acc128 is None else acc128 + term
    for a in range(dpj):
        acc = acc + acc128[:, d * a:d * (a + 1)]
    o_ref[...] = acc


def _node_update_body(accm_ref, acco_ref, out_ref, root_ref, cb_ref,
                      wih_ref, whh_ref, bih_ref, bhh_ref, hnew_ref, *, n):
    aggr = accm_ref[0, :n, :] + accm_ref[1, :n, :]
    deg = jnp.maximum(acco_ref[0, :n, 0:1] + acco_ref[1, :n, 0:1], 1.0)
    aggr = aggr / deg
    h = out_ref[...]
    d = h.shape[1]
    m = jnp.maximum(
        jnp.dot(h, root_ref[...], preferred_element_type=jnp.float32)
        + aggr + cb_ref[...], 0.0)
    gi = jnp.dot(m, wih_ref[...], preferred_element_type=jnp.float32) + bih_ref[...]
    gh = jnp.dot(h, whh_ref[...], preferred_element_type=jnp.float32) + bhh_ref[...]
    r = jax.nn.sigmoid(gi[:, :d] + gh[:, :d])
    z = jax.nn.sigmoid(gi[:, d:2 * d] + gh[:, d:2 * d])
    nn_ = jnp.tanh(gi[:, 2 * d:] + r * gh[:, 2 * d:])
    hnew_ref[...] = (1.0 - z) * nn_ + z * h


def _set2set_body(out_ref, brow_ref, wih_ref, whh_ref, bias_ref,
                  l1w_ref, l1b_ref, l2w_ref, l2b_ref, y_ref, *, nb, steps):
    xo = out_ref[...]
    n, d = xo.shape
    brow = brow_ref[...]
    gid = lax.broadcasted_iota(jnp.int32, (nb, n), 0)
    msk = gid == brow
    mf = msk.astype(jnp.float32)
    q_star = jnp.zeros((nb, 2 * d), jnp.float32)
    hh = jnp.zeros((nb, d), jnp.float32)
    cc = jnp.zeros((nb, d), jnp.float32)
    for _ in range(steps):
        gates = (jnp.dot(q_star, wih_ref[...], preferred_element_type=jnp.float32)
                 + jnp.dot(hh, whh_ref[...], preferred_element_type=jnp.float32)
                 + bias_ref[...])
        ii = jax.nn.sigmoid(gates[:, :d])
        ff = jax.nn.sigmoid(gates[:, d:2 * d])
        gg = jnp.tanh(gates[:, 2 * d:3 * d])
        oo = jax.nn.sigmoid(gates[:, 3 * d:])
        cc = ff * cc + ii * gg
        hh = oo * jnp.tanh(cc)
        s = lax.dot_general(hh, xo, (((1,), (1,)), ((), ())),
                            preferred_element_type=jnp.float32)
        e = jnp.sum(mf * s, axis=0, keepdims=True)
        emax = jnp.max(jnp.where(msk, e, -3e38), axis=1, keepdims=True)
        emax = jnp.where(emax > -1e38, emax, 0.0)
        eg = jnp.sum(mf * emax, axis=0, keepdims=True)
        ex = jnp.exp(e - eg)
        den = jnp.sum(mf * ex, axis=1, keepdims=True)
        dg = jnp.sum(mf * den, axis=0, keepdims=True)
        a = ex / jnp.maximum(dg, 1e-16)
        r_read = jnp.dot(mf * a, xo, preferred_element_type=jnp.float32)
        q_star = jnp.concatenate([hh, r_read], axis=1)
    y = jnp.maximum(
        jnp.dot(q_star, l1w_ref[...], preferred_element_type=jnp.float32)
        + l1b_ref[...], 0.0)
    y_ref[...] = (jnp.dot(y, l2w_ref[...], preferred_element_type=jnp.float32)
                  + l2b_ref[...])


def _tc_node_init(x, wt, b):
    n = x.shape[0]
    dout = wt.shape[1]
    return pl.pallas_call(
        _node_init_body,
        out_shape=jax.ShapeDtypeStruct((n, dout), jnp.float32),
    )(x, wt, b)


def _tc_h1(ea, w1t, b1, te):
    e_pad, fa = ea.shape
    c_h = w1t.shape[1]
    return pl.pallas_call(
        _h1_body,
        grid=(e_pad // te,),
        in_specs=[pl.BlockSpec((te, fa), lambda i: (i, 0)),
                  pl.BlockSpec((fa, c_h), lambda i: (0, 0)),
                  pl.BlockSpec((1, c_h), lambda i: (0, 0))],
        out_specs=pl.BlockSpec((te, c_h), lambda i: (i, 0)),
        out_shape=jax.ShapeDtypeStruct((e_pad, c_h), jnp.bfloat16),
    )(ea, w1t, b1)


def _tc_msg(xs, h1, w2t, b2r, te):
    e_pad, d = xs.shape
    c_h = h1.shape[1]
    return pl.pallas_call(
        _msg_body,
        grid=(e_pad // te,),
        in_specs=[pl.BlockSpec((te, d), lambda i: (i, 0)),
                  pl.BlockSpec((te, c_h), lambda i: (i, 0)),
                  pl.BlockSpec((c_h, d * d), lambda i: (0, 0)),
                  pl.BlockSpec((d, d), lambda i: (0, 0))],
        out_specs=pl.BlockSpec((te, d), lambda i: (i, 0)),
        out_shape=jax.ShapeDtypeStruct((e_pad, d), jnp.float32),
    )(xs, h1, w2t, b2r)


def _tc_node_update(accs, acc_ones, out, root, cb, wiht, whht, bih, bhh, n):
    d = out.shape[1]
    return pl.pallas_call(
        functools.partial(_node_update_body, n=n),
        out_shape=jax.ShapeDtypeStruct((n, d), jnp.float32),
    )(accs, acc_ones, out, root, cb, wiht, whht, bih, bhh)


def _tc_set2set(out, brow, wiht, whht, bias, l1wt, l1b, l2wt, l2b, nb, steps):
    return pl.pallas_call(
        functools.partial(_set2set_body, nb=nb, steps=steps),
        out_shape=jax.ShapeDtypeStruct((nb, 1), jnp.float32),
    )(out, brow, wiht, whht, bias, l1wt, l1b, l2wt, l2b)


def _pick_group(nch):
    gb = 10
    while nch % gb:
        gb -= 1
    return gb


def _sc_gather(table, idx3, e_pad, nch):
    d = table.shape[1]
    gb = _pick_group(nch)
    ng = nch // gb
    rpg = gb * _CH
    mesh = plsc.VectorSubcoreMesh(core_axis_name="c", subcore_axis_name="s")

    def body(tab_ref, idx_ref, out_ref, idx_v, buf0, buf1, sem0, sem1):
        c = lax.axis_index("c")
        s = lax.axis_index("s")
        wid = s * 2 + c
        pltpu.sync_copy(idx_ref.at[wid], idx_v)
        base = wid * nch * _CH
        bufs = (buf0, buf1)
        sems = (sem0, sem1)

        def fire(g, buf, sem):
            for a in range(gb):
                pltpu.async_copy(tab_ref.at[idx_v.at[g * gb + a]],
                                 buf.at[pl.ds(a * _CH, _CH)], sem)

        for g in range(min(2, ng)):
            fire(g, bufs[g % 2], sems[g % 2])
        for g in range(ng):
            buf = bufs[g % 2]
            sem = sems[g % 2]
            dst = out_ref.at[pl.ds(base + g * rpg, rpg)]
            pltpu.make_async_copy(dst, buf, sem).wait()
            pltpu.sync_copy(buf, dst)
            if g + 2 < ng:
                fire(g + 2, buf, sem)

    f = pl.kernel(body,
                  out_type=jax.ShapeDtypeStruct((e_pad, d), jnp.float32),
                  mesh=mesh,
                  compiler_params=pltpu.CompilerParams(use_tc_tiling_on_sc=False),
                  scratch_types=[pltpu.VMEM((nch, _CH), jnp.int32),
                                 pltpu.VMEM((rpg, d), jnp.float32),
                                 pltpu.VMEM((rpg, d), jnp.float32),
                                 pltpu.SemaphoreType.DMA,
                                 pltpu.SemaphoreType.DMA])
    return f(table, idx3)


def _sc_scatter(msg, idx3, zeros_acc, n_acc, nch):
    d = msg.shape[1]
    rp = n_acc // 16
    mesh = plsc.VectorSubcoreMesh(core_axis_name="c", subcore_axis_name="s")

    gb = _pick_group(nch)
    ng = nch // gb
    rpg = gb * _CH

    def body(msg_ref, idx_ref, z_ref, out_ref, idx_v, buf0, buf1, acc_sh,
             sem0, sem1):
        c = lax.axis_index("c")
        s = lax.axis_index("s")
        wid = s * 2 + c
        pltpu.sync_copy(z_ref.at[pl.ds(s * rp, rp)], acc_sh.at[pl.ds(s * rp, rp)])
        pltpu.sync_copy(idx_ref.at[wid], idx_v)
        plsc.subcore_barrier()
        base = wid * nch * _CH
        bufs = (buf0, buf1)
        sems = (sem0, sem1)

        for g in range(min(2, ng)):
            pltpu.async_copy(msg_ref.at[pl.ds(base + g * rpg, rpg)],
                             bufs[g % 2], sems[g % 2])
        for g in range(ng):
            buf = bufs[g % 2]
            sem = sems[g % 2]
            pltpu.make_async_copy(
                msg_ref.at[pl.ds(base + g * rpg, rpg)], buf, sem).wait()
            for a in range(gb):
                pltpu.sync_copy(buf.at[pl.ds(a * _CH, _CH)],
                                acc_sh.at[idx_v.at[g * gb + a]], add=True)
            if g + 2 < ng:
                pltpu.async_copy(msg_ref.at[pl.ds(base + (g + 2) * rpg, rpg)],
                                 buf, sem)
        plsc.subcore_barrier()
        pltpu.sync_copy(acc_sh.at[pl.ds(s * rp, rp)],
                        out_ref.at[c, pl.ds(s * rp, rp)])

    f = pl.kernel(body,
                  out_type=jax.ShapeDtypeStruct((2, n_acc, d), jnp.float32),
                  mesh=mesh,
                  compiler_params=pltpu.CompilerParams(use_tc_tiling_on_sc=False),
                  scratch_types=[pltpu.VMEM((nch, _CH), jnp.int32),
                                 pltpu.VMEM((rpg, d), jnp.float32),
                                 pltpu.VMEM((rpg, d), jnp.float32),
                                 pltpu.VMEM_SHARED((n_acc, d), jnp.float32),
                                 pltpu.SemaphoreType.DMA,
                                 pltpu.SemaphoreType.DMA])
    return f(msg, idx3, zeros_acc)


def kernel(x, edge_index, edge_attr, batch, lin0_w, lin0_b, mlp_w1, mlp_b1,
           mlp_w2, mlp_b2, conv_root, conv_bias, gru_w_ih, gru_w_hh, gru_b_ih,
           gru_b_hh, lstm_w_ih, lstm_w_hh, lstm_b_ih, lstm_b_hh, lin1_w,
           lin1_b, lin2_w, lin2_b):
    n, _ = x.shape
    e = edge_index.shape[1]
    d = lin0_w.shape[0]
    c_h = mlp_w1.shape[0]
    nb = _NB

    nch = -(-e // (_NW * _CH))
    e_pad = _NW * nch * _CH
    n_acc = ((n + 1) + 15) // 16 * 16

    src = edge_index[0]
    dst = edge_index[1]
    pad_e = e_pad - e
    src3 = jnp.concatenate([src, jnp.zeros((pad_e,), jnp.int32)]).reshape(
        _NW, nch, _CH)
    dst3 = jnp.concatenate([dst, jnp.full((pad_e,), n, jnp.int32)]).reshape(
        _NW, nch, _CH)
    ea_p = jnp.concatenate(
        [edge_attr, jnp.zeros((pad_e, edge_attr.shape[1]), jnp.float32)])
    zeros_acc = jnp.zeros((n_acc, d), jnp.float32)
    ones_msg = jnp.ones((e_pad, d), jnp.float32)
    brow = batch.reshape(1, n)

    out = _tc_node_init(x, lin0_w.T, lin0_b.reshape(1, d))
    h1 = _tc_h1(ea_p, mlp_w1.T, mlp_b1.reshape(1, c_h), _TE)
    acc_ones = _sc_scatter(ones_msg, dst3, zeros_acc, n_acc, nch)

    w2t = mlp_w2.T.astype(jnp.bfloat16)
    b2r = mlp_b2.reshape(d, d)
    wiht = gru_w_ih.T
    whht = gru_w_hh.T
    bih = gru_b_ih.reshape(1, 3 * d)
    bhh = gru_b_hh.reshape(1, 3 * d)
    for _ in range(3):
        xsrc = _sc_gather(out, src3, e_pad, nch)
        msg = _tc_msg(xsrc, h1, w2t, b2r, _TE)
        accs = _sc_scatter(msg, dst3, zeros_acc, n_acc, nch)
        out = _tc_node_update(accs, acc_ones, out, conv_root,
                              conv_bias.reshape(1, d), wiht, whht, bih, bhh, n)

    y = _tc_set2set(out, brow, lstm_w_ih.T, lstm_w_hh.T,
                    (lstm_b_ih + lstm_b_hh).reshape(1, 4 * d),
                    lin1_w.T, lin1_b.reshape(1, d), lin2_w.T,
                    lin2_b.reshape(1, 1), nb, 3)
    return y.reshape(-1)


# speedup vs baseline: 2.3730x; 2.0703x over previous
import functools

import jax
import jax.numpy as jnp
from jax import lax
from jax.experimental import pallas as pl
from jax.experimental.pallas import tpu as pltpu
from jax.experimental.pallas import tpu_sc as plsc

_NW = 32
_CH = 128
_TE = 2048
_NB = 64


def _node_init_body(x_ref, w_ref, b_ref, o_ref):
    o_ref[...] = jnp.maximum(
        jnp.dot(x_ref[...], w_ref[...], preferred_element_type=jnp.float32)
        + b_ref[...], 0.0)


def _h1_body(ea_ref, w_ref, b_ref, o_ref):
    o_ref[...] = jnp.maximum(
        jnp.dot(ea_ref[...], w_ref[...], preferred_element_type=jnp.float32)
        + b_ref[...], 0.0).astype(jnp.bfloat16)


def _msg_body(xs_ref, h1_ref, w2t_ref, b2r_ref, erep_ref, o_ref):
    te, d = xs_ref.shape
    w = jnp.dot(h1_ref[...], w2t_ref[...],
                preferred_element_type=jnp.float32).astype(jnp.bfloat16)
    xs = xs_ref[...]
    xrep = jnp.dot(xs, erep_ref[...],
                   preferred_element_type=jnp.float32).astype(jnp.bfloat16)
    p = (xrep * w).astype(jnp.float32)
    width = d * d
    while width > d:
        width //= 2
        p = p[:, :width] + p[:, width:]
    o_ref[...] = p + jnp.dot(xs, b2r_ref[...], preferred_element_type=jnp.float32)


def _node_update_body(accm_ref, acco_ref, out_ref, root_ref, cb_ref,
                      wih_ref, whh_ref, bih_ref, bhh_ref, hnew_ref, *, n):
    aggr = accm_ref[0, :n, :] + accm_ref[1, :n, :]
    deg = jnp.maximum(acco_ref[0, :n, 0:1] + acco_ref[1, :n, 0:1], 1.0)
    aggr = aggr / deg
    h = out_ref[...]
    d = h.shape[1]
    m = jnp.maximum(
        jnp.dot(h, root_ref[...], preferred_element_type=jnp.float32)
        + aggr + cb_ref[...], 0.0)
    gi = jnp.dot(m, wih_ref[...], preferred_element_type=jnp.float32) + bih_ref[...]
    gh = jnp.dot(h, whh_ref[...], preferred_element_type=jnp.float32) + bhh_ref[...]
    r = jax.nn.sigmoid(gi[:, :d] + gh[:, :d])
    z = jax.nn.sigmoid(gi[:, d:2 * d] + gh[:, d:2 * d])
    nn_ = jnp.tanh(gi[:, 2 * d:] + r * gh[:, 2 * d:])
    hnew_ref[...] = (1.0 - z) * nn_ + z * h


def _set2set_body(out_ref, brow_ref, wih_ref, whh_ref, bias_ref,
                  l1w_ref, l1b_ref, l2w_ref, l2b_ref, y_ref, *, nb, steps):
    xo = out_ref[...]
    n, d = xo.shape
    brow = brow_ref[...]
    gid = lax.broadcasted_iota(jnp.int32, (nb, n), 0)
    msk = gid == brow
    mf = msk.astype(jnp.float32)
    q_star = jnp.zeros((nb, 2 * d), jnp.float32)
    hh = jnp.zeros((nb, d), jnp.float32)
    cc = jnp.zeros((nb, d), jnp.float32)
    for _ in range(steps):
        gates = (jnp.dot(q_star, wih_ref[...], preferred_element_type=jnp.float32)
                 + jnp.dot(hh, whh_ref[...], preferred_element_type=jnp.float32)
                 + bias_ref[...])
        ii = jax.nn.sigmoid(gates[:, :d])
        ff = jax.nn.sigmoid(gates[:, d:2 * d])
        gg = jnp.tanh(gates[:, 2 * d:3 * d])
        oo = jax.nn.sigmoid(gates[:, 3 * d:])
        cc = ff * cc + ii * gg
        hh = oo * jnp.tanh(cc)
        s = lax.dot_general(hh, xo, (((1,), (1,)), ((), ())),
                            preferred_element_type=jnp.float32)
        e = jnp.sum(mf * s, axis=0, keepdims=True)
        emax = jnp.max(jnp.where(msk, e, -3e38), axis=1, keepdims=True)
        emax = jnp.where(emax > -1e38, emax, 0.0)
        eg = jnp.sum(mf * emax, axis=0, keepdims=True)
        ex = jnp.exp(e - eg)
        den = jnp.sum(mf * ex, axis=1, keepdims=True)
        dg = jnp.sum(mf * den, axis=0, keepdims=True)
        a = ex / jnp.maximum(dg, 1e-16)
        r_read = jnp.dot(mf * a, xo, preferred_element_type=jnp.float32)
        q_star = jnp.concatenate([hh, r_read], axis=1)
    y = jnp.maximum(
        jnp.dot(q_star, l1w_ref[...], preferred_element_type=jnp.float32)
        + l1b_ref[...], 0.0)
    y_ref[...] = (jnp.dot(y, l2w_ref[...], preferred_element_type=jnp.float32)
                  + l2b_ref[...])


def _tc_node_init(x, wt, b):
    n = x.shape[0]
    dout = wt.shape[1]
    return pl.pallas_call(
        _node_init_body,
        out_shape=jax.ShapeDtypeStruct((n, dout), jnp.float32),
    )(x, wt, b)


def _tc_h1(ea, w1t, b1, te):
    e_pad, fa = ea.shape
    c_h = w1t.shape[1]
    return pl.pallas_call(
        _h1_body,
        grid=(e_pad // te,),
        in_specs=[pl.BlockSpec((te, fa), lambda i: (i, 0)),
                  pl.BlockSpec((fa, c_h), lambda i: (0, 0)),
                  pl.BlockSpec((1, c_h), lambda i: (0, 0))],
        out_specs=pl.BlockSpec((te, c_h), lambda i: (i, 0)),
        out_shape=jax.ShapeDtypeStruct((e_pad, c_h), jnp.bfloat16),
    )(ea, w1t, b1)


def _tc_msg(xs, h1, w2t, b2r, erep, te):
    e_pad, d = xs.shape
    c_h = h1.shape[1]
    return pl.pallas_call(
        _msg_body,
        grid=(e_pad // te,),
        in_specs=[pl.BlockSpec((te, d), lambda i: (i, 0)),
                  pl.BlockSpec((te, c_h), lambda i: (i, 0)),
                  pl.BlockSpec((c_h, d * d), lambda i: (0, 0)),
                  pl.BlockSpec((d, d), lambda i: (0, 0)),
                  pl.BlockSpec((d, d * d), lambda i: (0, 0))],
        out_specs=pl.BlockSpec((te, d), lambda i: (i, 0)),
        out_shape=jax.ShapeDtypeStruct((e_pad, d), jnp.float32),
    )(xs, h1, w2t, b2r, erep)


def _tc_node_update(accs, acc_ones, out, root, cb, wiht, whht, bih, bhh, n):
    d = out.shape[1]
    return pl.pallas_call(
        functools.partial(_node_update_body, n=n),
        out_shape=jax.ShapeDtypeStruct((n, d), jnp.float32),
    )(accs, acc_ones, out, root, cb, wiht, whht, bih, bhh)


def _tc_set2set(out, brow, wiht, whht, bias, l1wt, l1b, l2wt, l2b, nb, steps):
    return pl.pallas_call(
        functools.partial(_set2set_body, nb=nb, steps=steps),
        out_shape=jax.ShapeDtypeStruct((nb, 1), jnp.float32),
    )(out, brow, wiht, whht, bias, l1wt, l1b, l2wt, l2b)


def _pick_group(nch):
    gb = 10
    while nch % gb:
        gb -= 1
    return gb


def _sc_gather(table, idx3, e_pad, nch):
    d = table.shape[1]
    gb = _pick_group(nch)
    ng = nch // gb
    rpg = gb * _CH
    mesh = plsc.VectorSubcoreMesh(core_axis_name="c", subcore_axis_name="s")

    def body(tab_ref, idx_ref, out_ref, idx_v, buf0, buf1, sem0, sem1):
        c = lax.axis_index("c")
        s = lax.axis_index("s")
        wid = s * 2 + c
        pltpu.sync_copy(idx_ref.at[wid], idx_v)
        base = wid * nch * _CH
        bufs = (buf0, buf1)
        sems = (sem0, sem1)

        def fire(g, buf, sem):
            for a in range(gb):
                pltpu.async_copy(tab_ref.at[idx_v.at[g * gb + a]],
                                 buf.at[pl.ds(a * _CH, _CH)], sem)

        for g in range(min(2, ng)):
            fire(g, bufs[g % 2], sems[g % 2])
        for g in range(ng):
            buf = bufs[g % 2]
            sem = sems[g % 2]
            dst = out_ref.at[pl.ds(base + g * rpg, rpg)]
            pltpu.make_async_copy(dst, buf, sem).wait()
            pltpu.sync_copy(buf, dst)
            if g + 2 < ng:
                fire(g + 2, buf, sem)

    f = pl.kernel(body,
                  out_type=jax.ShapeDtypeStruct((e_pad, d), jnp.float32),
                  mesh=mesh,
                  compiler_params=pltpu.CompilerParams(use_tc_tiling_on_sc=False),
                  scratch_types=[pltpu.VMEM((nch, _CH), jnp.int32),
                                 pltpu.VMEM((rpg, d), jnp.float32),
                                 pltpu.VMEM((rpg, d), jnp.float32),
                                 pltpu.SemaphoreType.DMA,
                                 pltpu.SemaphoreType.DMA])
    return f(table, idx3)


def _sc_scatter(msg, idx3, zeros_acc, n_acc, nch):
    d = msg.shape[1]
    rp = n_acc // 16
    mesh = plsc.VectorSubcoreMesh(core_axis_name="c", subcore_axis_name="s")

    gb = _pick_group(nch)
    ng = nch // gb
    rpg = gb * _CH

    def body(msg_ref, idx_ref, z_ref, out_ref, idx_v, buf0, buf1, acc_sh,
             sem0, sem1):
        c = lax.axis_index("c")
        s = lax.axis_index("s")
        wid = s * 2 + c
        pltpu.sync_copy(z_ref.at[pl.ds(s * rp, rp)], acc_sh.at[pl.ds(s * rp, rp)])
        pltpu.sync_copy(idx_ref.at[wid], idx_v)
        plsc.subcore_barrier()
        base = wid * nch * _CH
        bufs = (buf0, buf1)
        sems = (sem0, sem1)

        for g in range(min(2, ng)):
            pltpu.async_copy(msg_ref.at[pl.ds(base + g * rpg, rpg)],
                             bufs[g % 2], sems[g % 2])
        for g in range(ng):
            buf = bufs[g % 2]
            sem = sems[g % 2]
            pltpu.make_async_copy(
                msg_ref.at[pl.ds(base + g * rpg, rpg)], buf, sem).wait()
            for a in range(gb):
                pltpu.sync_copy(buf.at[pl.ds(a * _CH, _CH)],
                                acc_sh.at[idx_v.at[g * gb + a]], add=True)
            if g + 2 < ng:
                pltpu.async_copy(msg_ref.at[pl.ds(base + (g + 2) * rpg, rpg)],
                                 buf, sem)
        plsc.subcore_barrier()
        pltpu.sync_copy(acc_sh.at[pl.ds(s * rp, rp)],
                        out_ref.at[c, pl.ds(s * rp, rp)])

    f = pl.kernel(body,
                  out_type=jax.ShapeDtypeStruct((2, n_acc, d), jnp.float32),
                  mesh=mesh,
                  compiler_params=pltpu.CompilerParams(use_tc_tiling_on_sc=False),
                  scratch_types=[pltpu.VMEM((nch, _CH), jnp.int32),
                                 pltpu.VMEM((rpg, d), jnp.float32),
                                 pltpu.VMEM((rpg, d), jnp.float32),
                                 pltpu.VMEM_SHARED((n_acc, d), jnp.float32),
                                 pltpu.SemaphoreType.DMA,
                                 pltpu.SemaphoreType.DMA])
    return f(msg, idx3, zeros_acc)


def kernel(x, edge_index, edge_attr, batch, lin0_w, lin0_b, mlp_w1, mlp_b1,
           mlp_w2, mlp_b2, conv_root, conv_bias, gru_w_ih, gru_w_hh, gru_b_ih,
           gru_b_hh, lstm_w_ih, lstm_w_hh, lstm_b_ih, lstm_b_hh, lin1_w,
           lin1_b, lin2_w, lin2_b):
    n, _ = x.shape
    e = edge_index.shape[1]
    d = lin0_w.shape[0]
    c_h = mlp_w1.shape[0]
    nb = _NB

    nch = -(-e // (_NW * _CH))
    e_pad = _NW * nch * _CH
    n_acc = ((n + 1) + 15) // 16 * 16

    src = edge_index[0]
    dst = edge_index[1]
    pad_e = e_pad - e
    src3 = jnp.concatenate([src, jnp.zeros((pad_e,), jnp.int32)]).reshape(
        _NW, nch, _CH)
    dst3 = jnp.concatenate([dst, jnp.full((pad_e,), n, jnp.int32)]).reshape(
        _NW, nch, _CH)
    ea_p = jnp.concatenate(
        [edge_attr, jnp.zeros((pad_e, edge_attr.shape[1]), jnp.float32)])
    zeros_acc = jnp.zeros((n_acc, d), jnp.float32)
    ones_msg = jnp.ones((e_pad, d), jnp.float32)
    brow = batch.reshape(1, n)

    out = _tc_node_init(x, lin0_w.T, lin0_b.reshape(1, d))
    h1 = _tc_h1(ea_p, mlp_w1.T, mlp_b1.reshape(1, c_h), _TE)
    acc_ones = _sc_scatter(ones_msg, dst3, zeros_acc, n_acc, nch)

    w2t = mlp_w2.T.astype(jnp.bfloat16)
    b2r = mlp_b2.reshape(d, d)
    erep = jnp.kron(jnp.eye(d, dtype=jnp.float32),
                    jnp.ones((1, d), jnp.float32))
    wiht = gru_w_ih.T
    whht = gru_w_hh.T
    bih = gru_b_ih.reshape(1, 3 * d)
    bhh = gru_b_hh.reshape(1, 3 * d)
    for _ in range(3):
        xsrc = _sc_gather(out, src3, e_pad, nch)
        msg = _tc_msg(xsrc, h1, w2t, b2r, erep, _TE)
        accs = _sc_scatter(msg, dst3, zeros_acc, n_acc, nch)
        out = _tc_node_update(accs, acc_ones, out, conv_root,
                              conv_bias.reshape(1, d), wiht, whht, bih, bhh, n)

    y = _tc_set2set(out, brow, lstm_w_ih.T, lstm_w_hh.T,
                    (lstm_b_ih + lstm_b_hh).reshape(1, 4 * d),
                    lin1_w.T, lin1_b.reshape(1, d), lin2_w.T,
                    lin2_b.reshape(1, 1), nb, 3)
    return y.reshape(-1)


# speedup vs baseline: 2.3934x; 1.0086x over previous
import functools

import jax
import jax.numpy as jnp
from jax import lax
from jax.experimental import pallas as pl
from jax.experimental.pallas import tpu as pltpu
from jax.experimental.pallas import tpu_sc as plsc

_NW = 32
_CH = 128
_TE = 2048
_NB = 64


def _node_init_body(x_ref, w_ref, b_ref, o_ref):
    o_ref[...] = jnp.maximum(
        jnp.dot(x_ref[...], w_ref[...], preferred_element_type=jnp.float32)
        + b_ref[...], 0.0)


def _h1_body(ea_ref, w_ref, b_ref, o_ref):
    o_ref[...] = jnp.maximum(
        jnp.dot(ea_ref[...], w_ref[...], preferred_element_type=jnp.float32)
        + b_ref[...], 0.0).astype(jnp.bfloat16)


def _msg_body(xs_ref, ea_ref, w1t_ref, b1_ref, w2t_ref, b2r_ref, erep_ref,
              o_ref):
    te, d = xs_ref.shape
    h1 = jnp.maximum(
        jnp.dot(ea_ref[...], w1t_ref[...], preferred_element_type=jnp.float32)
        + b1_ref[...], 0.0).astype(jnp.bfloat16)
    w = jnp.dot(h1, w2t_ref[...],
                preferred_element_type=jnp.float32).astype(jnp.bfloat16)
    xs = xs_ref[...]
    xrep = jnp.dot(xs, erep_ref[...],
                   preferred_element_type=jnp.float32).astype(jnp.bfloat16)
    p = (xrep * w).astype(jnp.float32)
    width = d * d
    while width > d:
        width //= 2
        p = p[:, :width] + p[:, width:]
    o_ref[...] = p + jnp.dot(xs, b2r_ref[...], preferred_element_type=jnp.float32)


def _node_update_body(accm_ref, acco_ref, out_ref, root_ref, cb_ref,
                      wih_ref, whh_ref, bih_ref, bhh_ref, hnew_ref, *, n):
    aggr = accm_ref[0, :n, :] + accm_ref[1, :n, :]
    deg = jnp.maximum(acco_ref[0, :n, 0:1] + acco_ref[1, :n, 0:1], 1.0)
    aggr = aggr / deg
    h = out_ref[...]
    d = h.shape[1]
    m = jnp.maximum(
        jnp.dot(h, root_ref[...], preferred_element_type=jnp.float32)
        + aggr + cb_ref[...], 0.0)
    gi = jnp.dot(m, wih_ref[...], preferred_element_type=jnp.float32) + bih_ref[...]
    gh = jnp.dot(h, whh_ref[...], preferred_element_type=jnp.float32) + bhh_ref[...]
    r = jax.nn.sigmoid(gi[:, :d] + gh[:, :d])
    z = jax.nn.sigmoid(gi[:, d:2 * d] + gh[:, d:2 * d])
    nn_ = jnp.tanh(gi[:, 2 * d:] + r * gh[:, 2 * d:])
    hnew_ref[...] = (1.0 - z) * nn_ + z * h


def _set2set_body(out_ref, brow_ref, wih_ref, whh_ref, bias_ref,
                  l1w_ref, l1b_ref, l2w_ref, l2b_ref, y_ref, *, nb, steps):
    xo = out_ref[...]
    n, d = xo.shape
    brow = brow_ref[...]
    gid = lax.broadcasted_iota(jnp.int32, (nb, n), 0)
    msk = gid == brow
    mf = msk.astype(jnp.float32)
    q_star = jnp.zeros((nb, 2 * d), jnp.float32)
    hh = jnp.zeros((nb, d), jnp.float32)
    cc = jnp.zeros((nb, d), jnp.float32)
    for _ in range(steps):
        gates = (jnp.dot(q_star, wih_ref[...], preferred_element_type=jnp.float32)
                 + jnp.dot(hh, whh_ref[...], preferred_element_type=jnp.float32)
                 + bias_ref[...])
        ii = jax.nn.sigmoid(gates[:, :d])
        ff = jax.nn.sigmoid(gates[:, d:2 * d])
        gg = jnp.tanh(gates[:, 2 * d:3 * d])
        oo = jax.nn.sigmoid(gates[:, 3 * d:])
        cc = ff * cc + ii * gg
        hh = oo * jnp.tanh(cc)
        s = lax.dot_general(hh, xo, (((1,), (1,)), ((), ())),
                            preferred_element_type=jnp.float32)
        e = jnp.sum(mf * s, axis=0, keepdims=True)
        emax = jnp.max(jnp.where(msk, e, -3e38), axis=1, keepdims=True)
        emax = jnp.where(emax > -1e38, emax, 0.0)
        eg = jnp.sum(mf * emax, axis=0, keepdims=True)
        ex = jnp.exp(e - eg)
        den = jnp.sum(mf * ex, axis=1, keepdims=True)
        dg = jnp.sum(mf * den, axis=0, keepdims=True)
        a = ex / jnp.maximum(dg, 1e-16)
        r_read = jnp.dot(mf * a, xo, preferred_element_type=jnp.float32)
        q_star = jnp.concatenate([hh, r_read], axis=1)
    y = jnp.maximum(
        jnp.dot(q_star, l1w_ref[...], preferred_element_type=jnp.float32)
        + l1b_ref[...], 0.0)
    y_ref[...] = (jnp.dot(y, l2w_ref[...], preferred_element_type=jnp.float32)
                  + l2b_ref[...])


def _tc_node_init(x, wt, b):
    n = x.shape[0]
    dout = wt.shape[1]
    return pl.pallas_call(
        _node_init_body,
        out_shape=jax.ShapeDtypeStruct((n, dout), jnp.float32),
    )(x, wt, b)


def _tc_h1(ea, w1t, b1, te):
    e_pad, fa = ea.shape
    c_h = w1t.shape[1]
    return pl.pallas_call(
        _h1_body,
        grid=(e_pad // te,),
        in_specs=[pl.BlockSpec((te, fa), lambda i: (i, 0)),
                  pl.BlockSpec((fa, c_h), lambda i: (0, 0)),
                  pl.BlockSpec((1, c_h), lambda i: (0, 0))],
        out_specs=pl.BlockSpec((te, c_h), lambda i: (i, 0)),
        out_shape=jax.ShapeDtypeStruct((e_pad, c_h), jnp.bfloat16),
    )(ea, w1t, b1)


def _tc_msg(xs, ea, w1t, b1, w2t, b2r, erep, te):
    e_pad, d = xs.shape
    fa = ea.shape[1]
    c_h = w1t.shape[1]
    return pl.pallas_call(
        _msg_body,
        grid=(e_pad // te,),
        in_specs=[pl.BlockSpec((te, d), lambda i: (i, 0)),
                  pl.BlockSpec((te, fa), lambda i: (i, 0)),
                  pl.BlockSpec((fa, c_h), lambda i: (0, 0)),
                  pl.BlockSpec((1, c_h), lambda i: (0, 0)),
                  pl.BlockSpec((c_h, d * d), lambda i: (0, 0)),
                  pl.BlockSpec((d, d), lambda i: (0, 0)),
                  pl.BlockSpec((d, d * d), lambda i: (0, 0))],
        out_specs=pl.BlockSpec((te, d), lambda i: (i, 0)),
        out_shape=jax.ShapeDtypeStruct((e_pad, d), jnp.float32),
    )(xs, ea, w1t, b1, w2t, b2r, erep)


def _tc_node_update(accs, acc_ones, out, root, cb, wiht, whht, bih, bhh, n):
    d = out.shape[1]
    return pl.pallas_call(
        functools.partial(_node_update_body, n=n),
        out_shape=jax.ShapeDtypeStruct((n, d), jnp.float32),
    )(accs, acc_ones, out, root, cb, wiht, whht, bih, bhh)


def _tc_set2set(out, brow, wiht, whht, bias, l1wt, l1b, l2wt, l2b, nb, steps):
    return pl.pallas_call(
        functools.partial(_set2set_body, nb=nb, steps=steps),
        out_shape=jax.ShapeDtypeStruct((nb, 1), jnp.float32),
    )(out, brow, wiht, whht, bias, l1wt, l1b, l2wt, l2b)


def _pick_group(nch):
    gb = 10
    while nch % gb:
        gb -= 1
    return gb


def _sc_gather(table, idx3, e_pad, nch):
    d = table.shape[1]
    gb = _pick_group(nch)
    ng = nch // gb
    rpg = gb * _CH
    mesh = plsc.VectorSubcoreMesh(core_axis_name="c", subcore_axis_name="s")

    def body(tab_ref, idx_ref, out_ref, idx_v, buf0, buf1, sem0, sem1):
        c = lax.axis_index("c")
        s = lax.axis_index("s")
        wid = s * 2 + c
        pltpu.sync_copy(idx_ref.at[wid], idx_v)
        base = wid * nch * _CH
        bufs = (buf0, buf1)
        sems = (sem0, sem1)

        def fire(g, buf, sem):
            for a in range(gb):
                pltpu.async_copy(tab_ref.at[idx_v.at[g * gb + a]],
                                 buf.at[pl.ds(a * _CH, _CH)], sem)

        for g in range(min(2, ng)):
            fire(g, bufs[g % 2], sems[g % 2])
        for g in range(ng):
            buf = bufs[g % 2]
            sem = sems[g % 2]
            dst = out_ref.at[pl.ds(base + g * rpg, rpg)]
            pltpu.make_async_copy(dst, buf, sem).wait()
            pltpu.sync_copy(buf, dst)
            if g + 2 < ng:
                fire(g + 2, buf, sem)

    f = pl.kernel(body,
                  out_type=jax.ShapeDtypeStruct((e_pad, d), jnp.float32),
                  mesh=mesh,
                  compiler_params=pltpu.CompilerParams(use_tc_tiling_on_sc=False),
                  scratch_types=[pltpu.VMEM((nch, _CH), jnp.int32),
                                 pltpu.VMEM((rpg, d), jnp.float32),
                                 pltpu.VMEM((rpg, d), jnp.float32),
                                 pltpu.SemaphoreType.DMA,
                                 pltpu.SemaphoreType.DMA])
    return f(table, idx3)


def _sc_scatter(msg, idx3, zeros_acc, n_acc, nch):
    d = msg.shape[1]
    rp = n_acc // 16
    mesh = plsc.VectorSubcoreMesh(core_axis_name="c", subcore_axis_name="s")

    gb = _pick_group(nch)
    ng = nch // gb
    rpg = gb * _CH

    def body(msg_ref, idx_ref, z_ref, out_ref, idx_v, buf0, buf1, acc_sh,
             sem0, sem1):
        c = lax.axis_index("c")
        s = lax.axis_index("s")
        wid = s * 2 + c
        pltpu.sync_copy(z_ref.at[pl.ds(s * rp, rp)], acc_sh.at[pl.ds(s * rp, rp)])
        pltpu.sync_copy(idx_ref.at[wid], idx_v)
        plsc.subcore_barrier()
        base = wid * nch * _CH
        bufs = (buf0, buf1)
        sems = (sem0, sem1)

        for g in range(min(2, ng)):
            pltpu.async_copy(msg_ref.at[pl.ds(base + g * rpg, rpg)],
                             bufs[g % 2], sems[g % 2])
        for g in range(ng):
            buf = bufs[g % 2]
            sem = sems[g % 2]
            pltpu.make_async_copy(
                msg_ref.at[pl.ds(base + g * rpg, rpg)], buf, sem).wait()
            for a in range(gb):
                pltpu.sync_copy(buf.at[pl.ds(a * _CH, _CH)],
                                acc_sh.at[idx_v.at[g * gb + a]], add=True)
            if g + 2 < ng:
                pltpu.async_copy(msg_ref.at[pl.ds(base + (g + 2) * rpg, rpg)],
                                 buf, sem)
        plsc.subcore_barrier()
        pltpu.sync_copy(acc_sh.at[pl.ds(s * rp, rp)],
                        out_ref.at[c, pl.ds(s * rp, rp)])

    f = pl.kernel(body,
                  out_type=jax.ShapeDtypeStruct((2, n_acc, d), jnp.float32),
                  mesh=mesh,
                  compiler_params=pltpu.CompilerParams(use_tc_tiling_on_sc=False),
                  scratch_types=[pltpu.VMEM((nch, _CH), jnp.int32),
                                 pltpu.VMEM((rpg, d), jnp.float32),
                                 pltpu.VMEM((rpg, d), jnp.float32),
                                 pltpu.VMEM_SHARED((n_acc, d), jnp.float32),
                                 pltpu.SemaphoreType.DMA,
                                 pltpu.SemaphoreType.DMA])
    return f(msg, idx3, zeros_acc)


def kernel(x, edge_index, edge_attr, batch, lin0_w, lin0_b, mlp_w1, mlp_b1,
           mlp_w2, mlp_b2, conv_root, conv_bias, gru_w_ih, gru_w_hh, gru_b_ih,
           gru_b_hh, lstm_w_ih, lstm_w_hh, lstm_b_ih, lstm_b_hh, lin1_w,
           lin1_b, lin2_w, lin2_b):
    n, _ = x.shape
    e = edge_index.shape[1]
    d = lin0_w.shape[0]
    c_h = mlp_w1.shape[0]
    nb = _NB

    nch = -(-e // (_NW * _CH))
    e_pad = _NW * nch * _CH
    n_acc = ((n + 1) + 15) // 16 * 16

    src = edge_index[0]
    dst = edge_index[1]
    pad_e = e_pad - e
    src3 = jnp.concatenate([src, jnp.zeros((pad_e,), jnp.int32)]).reshape(
        _NW, nch, _CH)
    dst3 = jnp.concatenate([dst, jnp.full((pad_e,), n, jnp.int32)]).reshape(
        _NW, nch, _CH)
    ea_p = jnp.concatenate(
        [edge_attr, jnp.zeros((pad_e, edge_attr.shape[1]), jnp.float32)])
    zeros_acc = jnp.zeros((n_acc, d), jnp.float32)
    ones_msg = jnp.ones((e_pad, d), jnp.float32)
    brow = batch.reshape(1, n)

    out = _tc_node_init(x, lin0_w.T, lin0_b.reshape(1, d))
    acc_ones = _sc_scatter(ones_msg, dst3, zeros_acc, n_acc, nch)
    w1t = mlp_w1.T
    b1 = mlp_b1.reshape(1, c_h)

    w2t = mlp_w2.T.astype(jnp.bfloat16)
    b2r = mlp_b2.reshape(d, d)
    erep = jnp.kron(jnp.eye(d, dtype=jnp.float32),
                    jnp.ones((1, d), jnp.float32))
    wiht = gru_w_ih.T
    whht = gru_w_hh.T
    bih = gru_b_ih.reshape(1, 3 * d)
    bhh = gru_b_hh.reshape(1, 3 * d)
    for _ in range(3):
        xsrc = _sc_gather(out, src3, e_pad, nch)
        msg = _tc_msg(xsrc, ea_p, w1t, b1, w2t, b2r, erep, _TE)
        accs = _sc_scatter(msg, dst3, zeros_acc, n_acc, nch)
        out = _tc_node_update(accs, acc_ones, out, conv_root,
                              conv_bias.reshape(1, d), wiht, whht, bih, bhh, n)

    y = _tc_set2set(out, brow, lstm_w_ih.T, lstm_w_hh.T,
                    (lstm_b_ih + lstm_b_hh).reshape(1, 4 * d),
                    lin1_w.T, lin1_b.reshape(1, d), lin2_w.T,
                    lin2_b.reshape(1, 1), nb, 3)
    return y.reshape(-1)
